# baseline (speedup 1.0000x reference)
import jax
import jax.numpy as jnp
from jax.experimental import pallas as pl
from jax.experimental.pallas import tpu as pltpu

TOKEN_BLOCK = 512
NBUF = 4


def _make_body(tokens, dim, num_experts):
    def inner(x_ref, out_ref, w_ref):
        xh = x_ref[...].astype(jnp.bfloat16)
        logits = jnp.dot(xh, w_ref[...], preferred_element_type=jnp.float32)
        out_ref[...] = jax.nn.sigmoid(logits)

    def outer(x_hbm, w_ref, out_hbm):
        pipeline = pltpu.emit_pipeline(
            lambda x_blk, out_blk: inner(x_blk, out_blk, w_ref),
            grid=(tokens // TOKEN_BLOCK,),
            in_specs=[
                pl.BlockSpec((TOKEN_BLOCK, dim), lambda i: (i, 0),
                             pipeline_mode=pl.Buffered(NBUF, use_lookahead=True)),
            ],
            out_specs=[
                pl.BlockSpec((TOKEN_BLOCK, num_experts), lambda i: (i, 0)),
            ],
        )
        pipeline(x_hbm, out_hbm)

    return outer


@jax.jit
def kernel(x, router_weight):
    tokens, dim = x.shape
    num_experts = router_weight.shape[0]
    wt = router_weight.T.astype(jnp.bfloat16)

    return pl.pallas_call(
        _make_body(tokens, dim, num_experts),
        in_specs=[
            pl.BlockSpec(memory_space=pltpu.HBM),
            pl.BlockSpec(memory_space=pltpu.VMEM),
        ],
        out_specs=pl.BlockSpec(memory_space=pltpu.HBM),
        out_shape=jax.ShapeDtypeStruct((tokens, num_experts), jnp.float32),
    )(x, wt)


# speedup vs baseline: 1.0152x; 1.0152x over previous
import jax
import jax.numpy as jnp
from jax.experimental import pallas as pl
from jax.experimental.pallas import tpu as pltpu

TOKEN_BLOCK = 512


def _router_block(x_ref, w_ref, out_ref):
    xh = x_ref[...].astype(jnp.bfloat16)
    logits = jnp.dot(xh, w_ref[...], preferred_element_type=jnp.float32)
    out_ref[...] = jax.nn.sigmoid(logits)


@jax.jit
def kernel(x, router_weight):
    tokens, dim = x.shape
    num_experts = router_weight.shape[0]
    wt = router_weight.T.astype(jnp.bfloat16)

    grid = (tokens // TOKEN_BLOCK,)
    return pl.pallas_call(
        _router_block,
        grid=grid,
        in_specs=[
            pl.BlockSpec((TOKEN_BLOCK, dim), lambda i: (i, 0)),
            pl.BlockSpec((dim, num_experts), lambda i: (0, 0)),
        ],
        out_specs=pl.BlockSpec((TOKEN_BLOCK, num_experts), lambda i: (i, 0)),
        out_shape=jax.ShapeDtypeStruct((tokens, num_experts), jnp.float32),
        compiler_params=pltpu.CompilerParams(
            dimension_semantics=("parallel",),
        ),
    )(x, wt)
